# baseline (speedup 1.0000x reference)
import jax
import jax.numpy as jnp
from jax import lax
from jax.experimental import pallas as pl
from jax.experimental.pallas import tpu as pltpu
from jax.experimental.pallas import tpu_sc as plsc

K = 20000
NW = 32
ROWS_PER_W = 640
TAIL_ROWS = K - (NW - 1) * ROWS_PER_W


def _body(ti_hbm, tabt_hbm, out_hbm, ti_v, idx2_v, out_v, sem):
    nc = 2
    wid = lax.axis_index("s") * nc + lax.axis_index("c")
    base = wid * ROWS_PER_W

    def inner(tab_v):
        tab_cp = pltpu.async_copy(tabt_hbm, tab_v, sem)
        pltpu.sync_copy(ti_hbm.at[0], ti_v)

        def compose(j, carry):
            first = ti_v[pl.ds(base + j * 16, 16)]
            idx2_v[pl.ds(j * 16, 16)] = plsc.load_gather(ti_v, [first])
            return carry

        def value(j, carry):
            idx2 = idx2_v[pl.ds(j * 16, 16)]
            for c in range(4):
                cc = jnp.full((16,), c, jnp.int32)
                out_v[c, pl.ds(j * 16, 16)] = plsc.load_gather(
                    tab_v, [cc, idx2]
                )
            return carry

        @pl.when(wid < NW - 1)
        def _full():
            lax.fori_loop(0, ROWS_PER_W // 16, compose, 0, unroll=4)
            tab_cp.wait()
            lax.fori_loop(0, ROWS_PER_W // 16, value, 0, unroll=4)
            for c in range(4):
                pltpu.sync_copy(
                    out_v.at[c], out_hbm.at[c, pl.ds(base, ROWS_PER_W)]
                )

        @pl.when(wid == NW - 1)
        def _tail():
            lax.fori_loop(0, TAIL_ROWS // 16, compose, 0, unroll=4)
            tab_cp.wait()
            lax.fori_loop(0, TAIL_ROWS // 16, value, 0, unroll=4)
            for c in range(4):
                pltpu.sync_copy(
                    out_v.at[c, pl.ds(0, TAIL_ROWS)],
                    out_hbm.at[c, pl.ds(base, TAIL_ROWS)],
                )

    pl.run_scoped(inner, pltpu.VMEM((4, K), jnp.float32))


@jax.jit
def _run(ti, tabt):
    mesh = plsc.VectorSubcoreMesh(
        core_axis_name="c", subcore_axis_name="s", num_cores=2, num_subcores=16
    )
    f = pl.kernel(
        _body,
        out_type=jax.ShapeDtypeStruct((4, K), jnp.float32),
        mesh=mesh,
        scratch_types=[
            pltpu.VMEM((K,), jnp.int32),
            pltpu.VMEM((ROWS_PER_W,), jnp.int32),
            pltpu.VMEM((4, ROWS_PER_W), jnp.float32),
            pltpu.SemaphoreType.DMA,
        ],
        compiler_params=pltpu.CompilerParams(
            needs_layout_passes=False, use_tc_tiling_on_sc=False
        ),
    )
    return f(ti, tabt)


def kernel(batch_idx, topk_idx, box_regression):
    tabt = box_regression[0, :K, :].T
    out = _run(topk_idx.astype(jnp.int32), tabt)
    return out.T[None]


# speedup vs baseline: 1.1526x; 1.1526x over previous
import jax
import jax.numpy as jnp
from jax import lax
from jax.experimental import pallas as pl
from jax.experimental.pallas import tpu as pltpu
from jax.experimental.pallas import tpu_sc as plsc

K = 20000
NW = 32
CH = 2512
TAIL = K - 7 * CH


def _body(ti_hbm, tabt_hbm, out_hbm, ti_v, plane_v, out_v, sem):
    nc = 2
    wid = lax.axis_index("s") * nc + lax.axis_index("c")
    r = wid >> 2
    c = wid & 3
    base = r * CH
    plane_cp = pltpu.async_copy(tabt_hbm.at[c], plane_v, sem)
    pltpu.sync_copy(ti_hbm.at[0], ti_v)
    plane_cp.wait()

    def step(j, carry):
        first = ti_v[pl.ds(base + j * 16, 16)]
        idx2 = plsc.load_gather(ti_v, [first])
        out_v[pl.ds(j * 16, 16)] = plsc.load_gather(plane_v, [idx2])
        return carry

    @pl.when(r < 7)
    def _full():
        lax.fori_loop(0, CH // 16, step, 0, unroll=4)
        pltpu.sync_copy(out_v, out_hbm.at[c, pl.ds(base, CH)])

    @pl.when(r == 7)
    def _tail():
        lax.fori_loop(0, TAIL // 16, step, 0, unroll=4)
        pltpu.sync_copy(
            out_v.at[pl.ds(0, TAIL)], out_hbm.at[c, pl.ds(base, TAIL)]
        )


@jax.jit
def _run(ti, tabt):
    mesh = plsc.VectorSubcoreMesh(
        core_axis_name="c", subcore_axis_name="s", num_cores=2, num_subcores=16
    )
    f = pl.kernel(
        _body,
        out_type=jax.ShapeDtypeStruct((4, K), jnp.float32),
        mesh=mesh,
        scratch_types=[
            pltpu.VMEM((K,), jnp.int32),
            pltpu.VMEM((K,), jnp.float32),
            pltpu.VMEM((CH,), jnp.float32),
            pltpu.SemaphoreType.DMA,
        ],
        compiler_params=pltpu.CompilerParams(
            needs_layout_passes=False, use_tc_tiling_on_sc=False
        ),
    )
    return f(ti, tabt)


def kernel(batch_idx, topk_idx, box_regression):
    tabt = box_regression[0, :K, :].T
    out = _run(topk_idx.astype(jnp.int32), tabt)
    return out.T[None]


# speedup vs baseline: 1.1534x; 1.0007x over previous
import jax
import jax.numpy as jnp
from jax import lax
from jax.experimental import pallas as pl
from jax.experimental.pallas import tpu as pltpu
from jax.experimental.pallas import tpu_sc as plsc

K = 20000
NW = 32
CH = 2512
TAIL = K - 7 * CH


def _body(ti_hbm, tabt_hbm, out_hbm, ti_v, plane_v, out_v, sem):
    nc = 2
    wid = lax.axis_index("s") * nc + lax.axis_index("c")
    r = wid >> 2
    c = wid & 3
    base = r * CH
    plane_cp = pltpu.async_copy(tabt_hbm.at[c], plane_v, sem)
    pltpu.sync_copy(ti_hbm.at[0], ti_v)
    plane_cp.wait()

    def step(j, carry):
        first = ti_v[pl.ds(base + j * 16, 16)]
        idx2 = plsc.load_gather(ti_v, [first])
        out_v[pl.ds(j * 16, 16)] = plsc.load_gather(plane_v, [idx2])
        return carry

    @pl.when(r < 7)
    def _full():
        lax.fori_loop(0, CH // 16, step, 0, unroll=8)
        pltpu.sync_copy(out_v, out_hbm.at[c, pl.ds(base, CH)])

    @pl.when(r == 7)
    def _tail():
        lax.fori_loop(0, TAIL // 16, step, 0, unroll=8)
        pltpu.sync_copy(
            out_v.at[pl.ds(0, TAIL)], out_hbm.at[c, pl.ds(base, TAIL)]
        )


@jax.jit
def _run(ti, tabt):
    mesh = plsc.VectorSubcoreMesh(
        core_axis_name="c", subcore_axis_name="s", num_cores=2, num_subcores=16
    )
    f = pl.kernel(
        _body,
        out_type=jax.ShapeDtypeStruct((4, K), jnp.float32),
        mesh=mesh,
        scratch_types=[
            pltpu.VMEM((K,), jnp.int32),
            pltpu.VMEM((K,), jnp.float32),
            pltpu.VMEM((CH,), jnp.float32),
            pltpu.SemaphoreType.DMA,
        ],
        compiler_params=pltpu.CompilerParams(
            needs_layout_passes=False, use_tc_tiling_on_sc=False
        ),
    )
    return f(ti, tabt)


def kernel(batch_idx, topk_idx, box_regression):
    tabt = box_regression[0, :K, :].T
    out = _run(topk_idx.astype(jnp.int32), tabt)
    return out.T[None]


# speedup vs baseline: 1.2300x; 1.0665x over previous
import jax
import jax.numpy as jnp
from jax import lax
from jax.experimental import pallas as pl
from jax.experimental.pallas import tpu as pltpu
from jax.experimental.pallas import tpu_sc as plsc

K = 20000
NW = 32
CH = 2512
TAIL = K - 7 * CH


def _body(ti_hbm, tabt_hbm, out_hbm, ti_v, plane_v, out_v, sem):
    nc = 2
    wid = lax.axis_index("s") * nc + lax.axis_index("c")
    r = wid >> 2
    c = wid & 3
    base = r * CH
    plane_cp = pltpu.async_copy(tabt_hbm.at[c], plane_v, sem)
    pltpu.sync_copy(ti_hbm.at[0], ti_v)
    plane_cp.wait()

    def step(j):
        first = ti_v[pl.ds(base + j, 16)]
        idx2 = plsc.load_gather(ti_v, [first])
        out_v[pl.ds(j, 16)] = plsc.load_gather(plane_v, [idx2])

    @pl.when(r < 7)
    def _full():
        @plsc.parallel_loop(0, CH, step=16, unroll=8)
        def _(j):
            step(j)

        pltpu.sync_copy(out_v, out_hbm.at[c, pl.ds(base, CH)])

    @pl.when(r == 7)
    def _tail():
        @plsc.parallel_loop(0, TAIL, step=16, unroll=8)
        def _(j):
            step(j)

        pltpu.sync_copy(
            out_v.at[pl.ds(0, TAIL)], out_hbm.at[c, pl.ds(base, TAIL)]
        )


@jax.jit
def _run(ti, tabt):
    mesh = plsc.VectorSubcoreMesh(
        core_axis_name="c", subcore_axis_name="s", num_cores=2, num_subcores=16
    )
    f = pl.kernel(
        _body,
        out_type=jax.ShapeDtypeStruct((4, K), jnp.float32),
        mesh=mesh,
        scratch_types=[
            pltpu.VMEM((K,), jnp.int32),
            pltpu.VMEM((K,), jnp.float32),
            pltpu.VMEM((CH,), jnp.float32),
            pltpu.SemaphoreType.DMA,
        ],
        compiler_params=pltpu.CompilerParams(
            needs_layout_passes=False, use_tc_tiling_on_sc=False
        ),
    )
    return f(ti, tabt)


def kernel(batch_idx, topk_idx, box_regression):
    tabt = box_regression[0, :K, :].T
    out = _run(topk_idx.astype(jnp.int32), tabt)
    return out.T[None]
